# baseline (speedup 1.0000x reference)
import functools

import jax
import jax.numpy as jnp
from jax.experimental import pallas as pl

N_NODES_C = 10000
N_EDGES_C = 320000
B_GRAPHS_C = 100
NPG_C = 100
FG_EDGES_C = 1600
HID_C = 128
K_C = 32
N_LAYERS_C = 8

EDGE_BLK = 2560
NODE_BLK = 2000


def _node_matmul_body(h_ref, w_ref, o_ref, *, relu_in):
    x = h_ref[...]
    if relu_in:
        x = jnp.maximum(x, 0.0)
    o_ref[...] = jnp.dot(x, w_ref[...], preferred_element_type=jnp.float32)


def _node_matmul(h, w_cat, relu_in):
    n, _ = h.shape
    m = w_cat.shape[1]
    grid = (n // NODE_BLK,)
    return pl.pallas_call(
        functools.partial(_node_matmul_body, relu_in=relu_in),
        grid=grid,
        in_specs=[
            pl.BlockSpec((NODE_BLK, HID_C), lambda i: (i, 0)),
            pl.BlockSpec((HID_C, m), lambda i: (0, 0)),
        ],
        out_specs=pl.BlockSpec((NODE_BLK, m), lambda i: (i, 0)),
        out_shape=jax.ShapeDtypeStruct((n, m), jnp.float32),
    )(h, w_cat)


def _edge_body(e_ref, g_ref, w_ref, b_ref, a_ref, fo_ref, lo_ref):
    ef = jnp.dot(e_ref[...], w_ref[...], preferred_element_type=jnp.float32)
    f = ef + g_ref[...] + b_ref[...]
    fo = jnp.where(f >= 0.0, f, 0.2 * f)
    fo_ref[...] = fo
    lo = jnp.sum(fo * a_ref[...], axis=1)
    lo_ref[...] = lo.reshape(1, 1, EDGE_BLK)


def _edge_kernel(e, g, w, b, attn):
    grid = (N_EDGES_C // EDGE_BLK,)
    return pl.pallas_call(
        _edge_body,
        grid=grid,
        in_specs=[
            pl.BlockSpec((EDGE_BLK, HID_C), lambda i: (i, 0)),
            pl.BlockSpec((EDGE_BLK, HID_C), lambda i: (i, 0)),
            pl.BlockSpec((HID_C, HID_C), lambda i: (0, 0)),
            pl.BlockSpec((1, HID_C), lambda i: (0, 0)),
            pl.BlockSpec((1, HID_C), lambda i: (0, 0)),
        ],
        out_specs=[
            pl.BlockSpec((EDGE_BLK, HID_C), lambda i: (i, 0)),
            pl.BlockSpec((1, 1, EDGE_BLK), lambda i: (i, 0, 0)),
        ],
        out_shape=[
            jax.ShapeDtypeStruct((N_EDGES_C, HID_C), jnp.float32),
            jax.ShapeDtypeStruct((N_EDGES_C // EDGE_BLK, 1, EDGE_BLK), jnp.float32),
        ],
    )(e, g, w, b, attn)


def kernel(h_tokens, e_tokens, edge_index, fg_edge_index, token_emb, e_token_emb, W_ni, W_nj, W_fij, egat_bias, egat_attn, W_node, W_fwd, b_fwd, gat_W, gat_attn_l, gat_attn_r, gat_bias, W_lin, b_lin, W_cls, b_cls):
    src, dst = edge_index[0], edge_index[1]
    h = jax.nn.relu(token_emb[h_tokens])
    e = e_token_emb[e_tokens]

    w_cat = jnp.concatenate([W_ni, W_nj, W_node], axis=2)

    for i in range(N_LAYERS_C):
        xcat = _node_matmul(h, w_cat[i], relu_in=(i > 0))
        xi, xj, xn = xcat[:, :HID_C], xcat[:, HID_C:2 * HID_C], xcat[:, 2 * HID_C:]
        g = xi[src] + xj[dst]
        f_out, logits2d = _edge_kernel(e, g, W_fij[i], egat_bias[i][None, :],
                                       egat_attn[i][None, :])
        e = f_out
        logits = logits2d.reshape(-1)
        m = jax.ops.segment_max(logits, dst, num_segments=N_NODES_C)
        ex = jnp.exp(logits - m[dst])
        den = jax.ops.segment_sum(ex, dst, num_segments=N_NODES_C)
        a = ex / den[dst]
        msg = xn[src] * a[:, None]
        h = jax.ops.segment_sum(msg, dst, num_segments=N_NODES_C)

    h = jax.nn.relu(h)
    h = jax.nn.relu(_node_matmul(h, W_fwd, relu_in=False) + b_fwd)

    hs = jnp.sort(h, axis=-1)
    hb = hs.reshape(B_GRAPHS_C, NPG_C, HID_C)
    order = jnp.argsort(-hb[:, :, -1], axis=1)[:, :K_C]
    pooled = jnp.take_along_axis(hb, order[:, :, None], axis=1).reshape(B_GRAPHS_C, K_C * HID_C)

    ft = pooled @ gat_W
    el = jnp.sum(ft * gat_attn_l, axis=-1)
    er = jnp.sum(ft * gat_attn_r, axis=-1)
    fsrc, fdst = fg_edge_index[0], fg_edge_index[1]
    lg = jax.nn.leaky_relu(el[fsrc] + er[fdst], negative_slope=0.2)
    m = jax.ops.segment_max(lg, fdst, num_segments=B_GRAPHS_C)
    ex = jnp.exp(lg - m[fdst])
    den = jax.ops.segment_sum(ex, fdst, num_segments=B_GRAPHS_C)
    a = ex / den[fdst]
    rst = jax.ops.segment_sum(ft[fsrc] * a[:, None], fdst, num_segments=B_GRAPHS_C) + gat_bias
    h = jax.nn.relu(rst)
    h = jax.nn.relu(h @ W_lin + b_lin)
    out = h @ W_cls + b_cls
    return out.reshape(-1, 2)


# speedup vs baseline: 1.1601x; 1.1601x over previous
import functools

import jax
import jax.numpy as jnp
from jax import lax
from jax.experimental import pallas as pl
from jax.experimental.pallas import tpu as pltpu
from jax.experimental.pallas import tpu_sc as plsc

N_NODES_C = 10000
N_EDGES_C = 320000
B_GRAPHS_C = 100
NPG_C = 100
FG_EDGES_C = 1600
HID_C = 128
K_C = 32
N_LAYERS_C = 8

EDGE_BLK = 2560
NODE_BLK = 2000

SC_NC = 2
SC_NS = 16
SC_W = SC_NC * SC_NS
K1_EPW = N_EDGES_C // SC_W
K1_BATCH = 400
K1_ITERS = K1_EPW // K1_BATCH


def _sc_mesh():
    return plsc.VectorSubcoreMesh(core_axis_name="c", subcore_axis_name="s",
                                  num_cores=SC_NC, num_subcores=SC_NS)


def _k1_body(src_hbm, dst_hbm, xi_hbm, xj_hbm, g_hbm,
             src_v, dst_v, xi_rows, xj_rows, sem1, sem2):
    wid = lax.axis_index("s") * SC_NC + lax.axis_index("c")

    def it_body(it, carry):
        base = wid * K1_EPW + it * K1_BATCH
        pltpu.sync_copy(src_hbm.at[pl.ds(base, K1_BATCH)], src_v)
        pltpu.sync_copy(dst_hbm.at[pl.ds(base, K1_BATCH)], dst_v)
        cp1 = pltpu.async_copy(xi_hbm.at[src_v], xi_rows, sem1)
        cp2 = pltpu.async_copy(xj_hbm.at[dst_v], xj_rows, sem2)
        cp1.wait()
        cp2.wait()

        def row(r, c2):
            for j in range(HID_C // 16):
                sl = pl.ds(j * 16, 16)
                xi_rows[r, sl] = xi_rows[r, sl] + xj_rows[r, sl]
            return c2

        lax.fori_loop(0, K1_BATCH, row, 0)
        pltpu.sync_copy(xi_rows, g_hbm.at[pl.ds(base, K1_BATCH)])
        return carry

    lax.fori_loop(0, K1_ITERS, it_body, 0)


def _edge_gather_add(src, dst, xi, xj):
    return pl.kernel(
        _k1_body,
        out_type=jax.ShapeDtypeStruct((N_EDGES_C, HID_C), jnp.float32),
        mesh=_sc_mesh(),
        scratch_types=[
            pltpu.VMEM((K1_BATCH,), jnp.int32),
            pltpu.VMEM((K1_BATCH,), jnp.int32),
            pltpu.VMEM((K1_BATCH, HID_C), jnp.float32),
            pltpu.VMEM((K1_BATCH, HID_C), jnp.float32),
            pltpu.SemaphoreType.DMA,
            pltpu.SemaphoreType.DMA,
        ],
    )(src, dst, xi, xj)


def _node_matmul_body(h_ref, w_ref, o_ref, *, relu_in):
    x = h_ref[...]
    if relu_in:
        x = jnp.maximum(x, 0.0)
    o_ref[...] = jnp.dot(x, w_ref[...], preferred_element_type=jnp.float32)


def _node_matmul(h, w_cat, relu_in):
    n, _ = h.shape
    m = w_cat.shape[1]
    grid = (n // NODE_BLK,)
    return pl.pallas_call(
        functools.partial(_node_matmul_body, relu_in=relu_in),
        grid=grid,
        in_specs=[
            pl.BlockSpec((NODE_BLK, HID_C), lambda i: (i, 0)),
            pl.BlockSpec((HID_C, m), lambda i: (0, 0)),
        ],
        out_specs=pl.BlockSpec((NODE_BLK, m), lambda i: (i, 0)),
        out_shape=jax.ShapeDtypeStruct((n, m), jnp.float32),
    )(h, w_cat)


def _node3_body(h_ref, w_ref, xi_ref, xj_ref, xn_ref, *, relu_in):
    x = h_ref[...]
    if relu_in:
        x = jnp.maximum(x, 0.0)
    out = jnp.dot(x, w_ref[...], preferred_element_type=jnp.float32)
    xi_ref[...] = out[:, :HID_C]
    xj_ref[...] = out[:, HID_C:2 * HID_C]
    xn_ref[...] = out[:, 2 * HID_C:]


def _node_matmul3(h, w_cat, relu_in):
    n, _ = h.shape
    grid = (n // NODE_BLK,)
    ospec = pl.BlockSpec((NODE_BLK, HID_C), lambda i: (i, 0))
    oshape = jax.ShapeDtypeStruct((n, HID_C), jnp.float32)
    return pl.pallas_call(
        functools.partial(_node3_body, relu_in=relu_in),
        grid=grid,
        in_specs=[
            pl.BlockSpec((NODE_BLK, HID_C), lambda i: (i, 0)),
            pl.BlockSpec((HID_C, 3 * HID_C), lambda i: (0, 0)),
        ],
        out_specs=[ospec, ospec, ospec],
        out_shape=[oshape, oshape, oshape],
    )(h, w_cat)


def _edge_body(e_ref, g_ref, w_ref, b_ref, a_ref, fo_ref, lo_ref):
    ef = jnp.dot(e_ref[...], w_ref[...], preferred_element_type=jnp.float32)
    f = ef + g_ref[...] + b_ref[...]
    fo = jnp.where(f >= 0.0, f, 0.2 * f)
    fo_ref[...] = fo
    lo = jnp.sum(fo * a_ref[...], axis=1)
    lo_ref[...] = lo.reshape(1, 1, EDGE_BLK)


def _edge_kernel(e, g, w, b, attn):
    grid = (N_EDGES_C // EDGE_BLK,)
    return pl.pallas_call(
        _edge_body,
        grid=grid,
        in_specs=[
            pl.BlockSpec((EDGE_BLK, HID_C), lambda i: (i, 0)),
            pl.BlockSpec((EDGE_BLK, HID_C), lambda i: (i, 0)),
            pl.BlockSpec((HID_C, HID_C), lambda i: (0, 0)),
            pl.BlockSpec((1, HID_C), lambda i: (0, 0)),
            pl.BlockSpec((1, HID_C), lambda i: (0, 0)),
        ],
        out_specs=[
            pl.BlockSpec((EDGE_BLK, HID_C), lambda i: (i, 0)),
            pl.BlockSpec((1, 1, EDGE_BLK), lambda i: (i, 0, 0)),
        ],
        out_shape=[
            jax.ShapeDtypeStruct((N_EDGES_C, HID_C), jnp.float32),
            jax.ShapeDtypeStruct((N_EDGES_C // EDGE_BLK, 1, EDGE_BLK), jnp.float32),
        ],
    )(e, g, w, b, attn)


def kernel(h_tokens, e_tokens, edge_index, fg_edge_index, token_emb, e_token_emb, W_ni, W_nj, W_fij, egat_bias, egat_attn, W_node, W_fwd, b_fwd, gat_W, gat_attn_l, gat_attn_r, gat_bias, W_lin, b_lin, W_cls, b_cls):
    src, dst = edge_index[0], edge_index[1]
    h = jax.nn.relu(token_emb[h_tokens])
    e = e_token_emb[e_tokens]

    w_cat = jnp.concatenate([W_ni, W_nj, W_node], axis=2)

    for i in range(N_LAYERS_C):
        xi, xj, xn = _node_matmul3(h, w_cat[i], relu_in=(i > 0))
        g = _edge_gather_add(src, dst, xi, xj)
        f_out, logits2d = _edge_kernel(e, g, W_fij[i], egat_bias[i][None, :],
                                       egat_attn[i][None, :])
        e = f_out
        logits = logits2d.reshape(-1)
        m = jax.ops.segment_max(logits, dst, num_segments=N_NODES_C)
        ex = jnp.exp(logits - m[dst])
        den = jax.ops.segment_sum(ex, dst, num_segments=N_NODES_C)
        a = ex / den[dst]
        msg = xn[src] * a[:, None]
        h = jax.ops.segment_sum(msg, dst, num_segments=N_NODES_C)

    h = jax.nn.relu(h)
    h = jax.nn.relu(_node_matmul(h, W_fwd, relu_in=False) + b_fwd)

    hs = jnp.sort(h, axis=-1)
    hb = hs.reshape(B_GRAPHS_C, NPG_C, HID_C)
    order = jnp.argsort(-hb[:, :, -1], axis=1)[:, :K_C]
    pooled = jnp.take_along_axis(hb, order[:, :, None], axis=1).reshape(B_GRAPHS_C, K_C * HID_C)

    ft = pooled @ gat_W
    el = jnp.sum(ft * gat_attn_l, axis=-1)
    er = jnp.sum(ft * gat_attn_r, axis=-1)
    fsrc, fdst = fg_edge_index[0], fg_edge_index[1]
    lg = jax.nn.leaky_relu(el[fsrc] + er[fdst], negative_slope=0.2)
    m = jax.ops.segment_max(lg, fdst, num_segments=B_GRAPHS_C)
    ex = jnp.exp(lg - m[fdst])
    den = jax.ops.segment_sum(ex, fdst, num_segments=B_GRAPHS_C)
    a = ex / den[fdst]
    rst = jax.ops.segment_sum(ft[fsrc] * a[:, None], fdst, num_segments=B_GRAPHS_C) + gat_bias
    h = jax.nn.relu(rst)
    h = jax.nn.relu(h @ W_lin + b_lin)
    out = h @ W_cls + b_cls
    return out.reshape(-1, 2)


# speedup vs baseline: 2.3089x; 1.9902x over previous
import functools

import jax
import jax.numpy as jnp
from jax import lax
from jax.experimental import pallas as pl
from jax.experimental.pallas import tpu as pltpu
from jax.experimental.pallas import tpu_sc as plsc

N_NODES_C = 10000
N_EDGES_C = 320000
B_GRAPHS_C = 100
NPG_C = 100
FG_EDGES_C = 1600
HID_C = 128
K_C = 32
N_LAYERS_C = 8

EDGE_BLK = 2560
NODE_BLK = 2000

SC_NC = 2
SC_NS = 16
SC_W = SC_NC * SC_NS
K1_EPW = N_EDGES_C // SC_W
K1_BATCH = 400
K1_ITERS = K1_EPW // K1_BATCH


def _sc_mesh():
    return plsc.VectorSubcoreMesh(core_axis_name="c", subcore_axis_name="s",
                                  num_cores=SC_NC, num_subcores=SC_NS)


def _k1_body(src_hbm, dst_hbm, xi_hbm, xj_hbm, g_hbm,
             src_v, dst_v, xi_rows, xj_rows, sem1, sem2):
    wid = lax.axis_index("s") * SC_NC + lax.axis_index("c")

    def it_body(it, carry):
        base = wid * K1_EPW + it * K1_BATCH
        pltpu.sync_copy(src_hbm.at[pl.ds(base, K1_BATCH)], src_v)
        pltpu.sync_copy(dst_hbm.at[pl.ds(base, K1_BATCH)], dst_v)
        cp1 = pltpu.async_copy(xi_hbm.at[src_v], xi_rows, sem1)
        cp2 = pltpu.async_copy(xj_hbm.at[dst_v], xj_rows, sem2)
        cp1.wait()
        cp2.wait()

        def row(r, c2):
            for j in range(HID_C // 16):
                sl = pl.ds(j * 16, 16)
                xi_rows[r, sl] = xi_rows[r, sl] + xj_rows[r, sl]
            return c2

        lax.fori_loop(0, K1_BATCH, row, 0)
        pltpu.sync_copy(xi_rows, g_hbm.at[pl.ds(base, K1_BATCH)])
        return carry

    lax.fori_loop(0, K1_ITERS, it_body, 0)


def _edge_gather_add(src, dst, xi, xj):
    return pl.kernel(
        _k1_body,
        out_type=jax.ShapeDtypeStruct((N_EDGES_C, HID_C), jnp.float32),
        mesh=_sc_mesh(),
        scratch_types=[
            pltpu.VMEM((K1_BATCH,), jnp.int32),
            pltpu.VMEM((K1_BATCH,), jnp.int32),
            pltpu.VMEM((K1_BATCH, HID_C), jnp.float32),
            pltpu.VMEM((K1_BATCH, HID_C), jnp.float32),
            pltpu.SemaphoreType.DMA,
            pltpu.SemaphoreType.DMA,
        ],
    )(src, dst, xi, xj)


def _fwd_body(ha_ref, hb_ref, w_ref, b_ref, o_ref):
    x = jnp.maximum(ha_ref[...] + hb_ref[...], 0.0)
    y = jnp.dot(x, w_ref[...], preferred_element_type=jnp.float32) + b_ref[...]
    o_ref[...] = jnp.maximum(y, 0.0)


def _fwd_matmul(ha, hb, w, b):
    n, _ = ha.shape
    grid = (n // NODE_BLK,)
    hspec = pl.BlockSpec((NODE_BLK, HID_C), lambda i: (i, 0))
    return pl.pallas_call(
        _fwd_body,
        grid=grid,
        in_specs=[
            hspec, hspec,
            pl.BlockSpec((HID_C, HID_C), lambda i: (0, 0)),
            pl.BlockSpec((1, HID_C), lambda i: (0, 0)),
        ],
        out_specs=pl.BlockSpec((NODE_BLK, HID_C), lambda i: (i, 0)),
        out_shape=jax.ShapeDtypeStruct((n, HID_C), jnp.float32),
    )(ha, hb, w, b)


def _node3_body1(h_ref, w_ref, xi_ref, xj_ref, xn_ref):
    out = jnp.dot(h_ref[...], w_ref[...], preferred_element_type=jnp.float32)
    xi_ref[...] = out[:, :HID_C]
    xj_ref[...] = out[:, HID_C:2 * HID_C]
    xn_ref[...] = out[:, 2 * HID_C:]


def _node3_body2(ha_ref, hb_ref, w_ref, xi_ref, xj_ref, xn_ref):
    x = jnp.maximum(ha_ref[...] + hb_ref[...], 0.0)
    out = jnp.dot(x, w_ref[...], preferred_element_type=jnp.float32)
    xi_ref[...] = out[:, :HID_C]
    xj_ref[...] = out[:, HID_C:2 * HID_C]
    xn_ref[...] = out[:, 2 * HID_C:]


def _node_matmul3(hs, w_cat):
    n = hs[0].shape[0]
    grid = (n // NODE_BLK,)
    hspec = pl.BlockSpec((NODE_BLK, HID_C), lambda i: (i, 0))
    ospec = pl.BlockSpec((NODE_BLK, HID_C), lambda i: (i, 0))
    oshape = jax.ShapeDtypeStruct((n, HID_C), jnp.float32)
    body = _node3_body1 if len(hs) == 1 else _node3_body2
    return pl.pallas_call(
        body,
        grid=grid,
        in_specs=[hspec] * len(hs) + [pl.BlockSpec((HID_C, 3 * HID_C), lambda i: (0, 0))],
        out_specs=[ospec, ospec, ospec],
        out_shape=[oshape, oshape, oshape],
    )(*hs, w_cat)


def _edge_body(e_ref, g_ref, w_ref, b_ref, a_ref, fo_ref, lo_ref, bm_ref):
    ef = jnp.dot(e_ref[...], w_ref[...], preferred_element_type=jnp.float32)
    f = ef + g_ref[...] + b_ref[...]
    fo = jnp.where(f >= 0.0, f, 0.2 * f)
    fo_ref[...] = fo
    lo = fo * a_ref[...]
    lo = jnp.sum(lo, axis=1)
    lo_ref[...] = lo.reshape(1, 1, EDGE_BLK)
    bm_ref[...] = jnp.max(lo.reshape(EDGE_BLK // HID_C, HID_C), axis=0).reshape(1, 1, HID_C)


def _edge_kernel(e, g, w, b, attn):
    grid = (N_EDGES_C // EDGE_BLK,)
    return pl.pallas_call(
        _edge_body,
        grid=grid,
        in_specs=[
            pl.BlockSpec((EDGE_BLK, HID_C), lambda i: (i, 0)),
            pl.BlockSpec((EDGE_BLK, HID_C), lambda i: (i, 0)),
            pl.BlockSpec((HID_C, HID_C), lambda i: (0, 0)),
            pl.BlockSpec((1, HID_C), lambda i: (0, 0)),
            pl.BlockSpec((1, HID_C), lambda i: (0, 0)),
        ],
        out_specs=[
            pl.BlockSpec((EDGE_BLK, HID_C), lambda i: (i, 0)),
            pl.BlockSpec((1, 1, EDGE_BLK), lambda i: (i, 0, 0)),
            pl.BlockSpec((1, 1, HID_C), lambda i: (i, 0, 0)),
        ],
        out_shape=[
            jax.ShapeDtypeStruct((N_EDGES_C, HID_C), jnp.float32),
            jax.ShapeDtypeStruct((N_EDGES_C // EDGE_BLK, 1, EDGE_BLK), jnp.float32),
            jax.ShapeDtypeStruct((N_EDGES_C // EDGE_BLK, 1, HID_C), jnp.float32),
        ],
    )(e, g, w, b, attn)


EC_C = N_EDGES_C // EDGE_BLK
ND_C = 10
NBLK_C = N_NODES_C // ND_C
DEN_SUB = 256


def _ex_body(lo_ref, bm_ref, ex_ref):
    big_l = jnp.max(bm_ref[...])
    ex_ref[...] = jnp.exp(lo_ref[...] - big_l)


def _ex_kernel(lo3d, bmax):
    return pl.pallas_call(
        _ex_body,
        grid=(EC_C,),
        in_specs=[
            pl.BlockSpec((1, 1, EDGE_BLK), lambda i: (i, 0, 0)),
            pl.BlockSpec((EC_C, 1, HID_C), lambda i: (0, 0, 0)),
        ],
        out_specs=pl.BlockSpec((1, 1, EDGE_BLK), lambda i: (i, 0, 0)),
        out_shape=jax.ShapeDtypeStruct((EC_C, 1, EDGE_BLK), jnp.float32),
    )(lo3d, bmax)


def _den_body(ex_ref, dst_ref, den_ref):
    n = pl.program_id(0)
    e = pl.program_id(1)

    @pl.when(e == 0)
    def _():
        den_ref[...] = jnp.zeros_like(den_ref)

    nodes = NBLK_C * n + jax.lax.broadcasted_iota(jnp.int32, (NBLK_C, 1), 0)
    acc = jnp.zeros((1, NBLK_C), jnp.float32)
    for k in range(EDGE_BLK // DEN_SUB):
        d_sub = dst_ref[0, 0, pl.ds(k * DEN_SUB, DEN_SUB)].reshape(1, DEN_SUB)
        ex_sub = ex_ref[0, 0, pl.ds(k * DEN_SUB, DEN_SUB)].reshape(1, DEN_SUB)
        masked = jnp.where(d_sub == nodes, ex_sub, 0.0)
        acc = acc + jnp.sum(masked, axis=1).reshape(1, NBLK_C)
    den_ref[...] = den_ref[...] + acc.reshape(1, 1, NBLK_C)


def _den_kernel(ex3d, dst3d):
    return pl.pallas_call(
        _den_body,
        grid=(ND_C, EC_C),
        in_specs=[
            pl.BlockSpec((1, 1, EDGE_BLK), lambda n, e: (e, 0, 0)),
            pl.BlockSpec((1, 1, EDGE_BLK), lambda n, e: (e, 0, 0)),
        ],
        out_specs=pl.BlockSpec((1, 1, NBLK_C), lambda n, e: (n, 0, 0)),
        out_shape=jax.ShapeDtypeStruct((ND_C, 1, NBLK_C), jnp.float32),
    )(ex3d, dst3d)


K3_BATCH = 80
K3_ITERS = K1_EPW // K3_BATCH
NODES_PER_TILE = N_NODES_C // SC_NS


def _k3_body(src_hbm, dst_hbm, ex_hbm, den_hbm, xn_hbm, hpart_hbm,
             src_v, dst_v, ex_v, den_v, rows, zbuf, sem, shared):
    cid = lax.axis_index("c")
    sid = lax.axis_index("s")
    wid = sid * SC_NC + cid

    def zero16(r, carry):
        for j in range(HID_C // 16):
            zbuf[r, pl.ds(j * 16, 16)] = jnp.zeros((16,), jnp.float32)
        return carry

    if True:
        lax.fori_loop(0, zbuf.shape[0], zero16, 0)

        @pl.when(sid < 5)
        def _():
            for k in range(10):
                pltpu.sync_copy(zbuf, shared.at[pl.ds(sid * 2000 + k * 200, 200)])

        plsc.subcore_barrier()

        def it_body(it, carry):
            base = wid * K1_EPW + it * K3_BATCH
            pltpu.sync_copy(src_hbm.at[pl.ds(base, K3_BATCH)], src_v)
            pltpu.sync_copy(dst_hbm.at[pl.ds(base, K3_BATCH)], dst_v)
            pltpu.sync_copy(ex_hbm.at[pl.ds(base, K3_BATCH)], ex_v)
            cp = pltpu.async_copy(den_hbm.at[dst_v], den_v, sem)
            cp.wait()
            cp = pltpu.async_copy(xn_hbm.at[src_v], rows, sem)
            cp.wait()
            for q in range(K3_BATCH // 16):
                sl = pl.ds(q * 16, 16)
                ex_v[sl] = ex_v[sl] / den_v[sl]

            def scale_group(q, c2):
                a16 = ex_v[pl.ds(q * 16, 16)]
                for l in range(16):
                    s = a16[l]
                    r = q * 16 + l
                    for j in range(HID_C // 16):
                        sl2 = pl.ds(j * 16, 16)
                        rows[r, sl2] = rows[r, sl2] * s
                return c2

            lax.fori_loop(0, K3_BATCH // 16, scale_group, 0)
            pltpu.sync_copy(rows, shared.at[dst_v], add=True)
            return carry

        lax.fori_loop(0, K3_ITERS, it_body, 0)
        plsc.subcore_barrier()

        @pl.when(sid < 5)
        def _():
            pltpu.sync_copy(
                shared.at[pl.ds(sid * 2000, 2000)],
                hpart_hbm.at[cid, pl.ds(sid * 2000, 2000)])


def _aggregate(src, dst, ex, den, xn):
    return pl.kernel(
        _k3_body,
        out_type=jax.ShapeDtypeStruct((SC_NC, N_NODES_C, HID_C), jnp.float32),
        mesh=_sc_mesh(),
        scratch_types=[
            pltpu.VMEM((K3_BATCH,), jnp.int32),
            pltpu.VMEM((K3_BATCH,), jnp.int32),
            pltpu.VMEM((K3_BATCH,), jnp.float32),
            pltpu.VMEM((K3_BATCH,), jnp.float32),
            pltpu.VMEM((K3_BATCH, HID_C), jnp.float32),
            pltpu.VMEM((200, HID_C), jnp.float32),
            pltpu.SemaphoreType.DMA,
            pltpu.VMEM_SHARED((N_NODES_C, HID_C), jnp.float32),
        ],
    )(src, dst, ex, den, xn)


def kernel(h_tokens, e_tokens, edge_index, fg_edge_index, token_emb, e_token_emb, W_ni, W_nj, W_fij, egat_bias, egat_attn, W_node, W_fwd, b_fwd, gat_W, gat_attn_l, gat_attn_r, gat_bias, W_lin, b_lin, W_cls, b_cls):
    src, dst = edge_index[0], edge_index[1]
    h = jax.nn.relu(token_emb[h_tokens])
    e = e_token_emb[e_tokens]
    dst3d = dst.reshape(EC_C, 1, EDGE_BLK)

    w_cat = jnp.concatenate([W_ni, W_nj, W_node], axis=2)

    hs = (h,)
    for i in range(N_LAYERS_C):
        xi, xj, xn = _node_matmul3(hs, w_cat[i])
        g = _edge_gather_add(src, dst, xi, xj)
        f_out, lo3d, bmax = _edge_kernel(e, g, W_fij[i], egat_bias[i][None, :],
                                         egat_attn[i][None, :])
        e = f_out
        ex3d = _ex_kernel(lo3d, bmax)
        den3d = _den_kernel(ex3d, dst3d)
        hpart = _aggregate(src, dst, ex3d.reshape(-1), den3d.reshape(-1), xn)
        hs = (hpart[0], hpart[1])

    h = _fwd_matmul(hs[0], hs[1], W_fwd, b_fwd[None, :])

    hs = jnp.sort(h, axis=-1)
    hb = hs.reshape(B_GRAPHS_C, NPG_C, HID_C)
    order = jnp.argsort(-hb[:, :, -1], axis=1)[:, :K_C]
    pooled = jnp.take_along_axis(hb, order[:, :, None], axis=1).reshape(B_GRAPHS_C, K_C * HID_C)

    ft = pooled @ gat_W
    el = jnp.sum(ft * gat_attn_l, axis=-1)
    er = jnp.sum(ft * gat_attn_r, axis=-1)
    fsrc, fdst = fg_edge_index[0], fg_edge_index[1]
    lg = jax.nn.leaky_relu(el[fsrc] + er[fdst], negative_slope=0.2)
    m = jax.ops.segment_max(lg, fdst, num_segments=B_GRAPHS_C)
    ex = jnp.exp(lg - m[fdst])
    den = jax.ops.segment_sum(ex, fdst, num_segments=B_GRAPHS_C)
    a = ex / den[fdst]
    rst = jax.ops.segment_sum(ft[fsrc] * a[:, None], fdst, num_segments=B_GRAPHS_C) + gat_bias
    h = jax.nn.relu(rst)
    h = jax.nn.relu(h @ W_lin + b_lin)
    out = h @ W_cls + b_cls
    return out.reshape(-1, 2)


# speedup vs baseline: 8.3695x; 3.6249x over previous
import functools

import jax
import jax.numpy as jnp
from jax import lax
from jax.experimental import pallas as pl
from jax.experimental.pallas import tpu as pltpu
from jax.experimental.pallas import tpu_sc as plsc

N_NODES_C = 10000
N_EDGES_C = 320000
B_GRAPHS_C = 100
NPG_C = 100
FG_EDGES_C = 1600
HID_C = 128
K_C = 32
N_LAYERS_C = 8

EDGE_BLK = 2560
NODE_BLK = 2000

SC_NC = 2
SC_NS = 16
SC_W = SC_NC * SC_NS
K1_EPW = N_EDGES_C // SC_W
K1_BATCH = 200
K1_ITERS = K1_EPW // K1_BATCH
K1_ROUNDS = (K1_ITERS + 1) // 2


def _sc_mesh():
    return plsc.VectorSubcoreMesh(core_axis_name="c", subcore_axis_name="s",
                                  num_cores=SC_NC, num_subcores=SC_NS)


def _k1_body(src_hbm, dst_hbm, xi_hbm, xj_hbm, g_hbm,
             srcv0, srcv1, dstv0, dstv1, a0, a1, b0, b1, sg0, sg1):
    wid = lax.axis_index("s") * SC_NC + lax.axis_index("c")
    srcv = (srcv0, srcv1)
    dstv = (dstv0, dstv1)
    abuf = (a0, a1)
    bbuf = (b0, b1)
    sg = (sg0, sg1)

    def issue(slot, it):
        base = wid * K1_EPW + it * K1_BATCH
        pltpu.sync_copy(src_hbm.at[pl.ds(base, K1_BATCH)], srcv[slot])
        pltpu.sync_copy(dst_hbm.at[pl.ds(base, K1_BATCH)], dstv[slot])
        pltpu.async_copy(xi_hbm.at[srcv[slot]], abuf[slot], sg[slot])
        pltpu.async_copy(xj_hbm.at[dstv[slot]], bbuf[slot], sg[slot])

    issue(0, 0)
    issue(1, 1)

    def round_body(r, carry):
        for slot in (0, 1):
            it = r * 2 + slot
            pltpu.make_async_copy(xi_hbm.at[srcv[slot]], abuf[slot],
                                  sg[slot]).wait()
            pltpu.make_async_copy(xj_hbm.at[dstv[slot]], bbuf[slot],
                                  sg[slot]).wait()

            def row(rr, c2, slot=slot):
                for j in range(HID_C // 16):
                    sl = pl.ds(j * 16, 16)
                    abuf[slot][rr, sl] = abuf[slot][rr, sl] + bbuf[slot][rr, sl]
                return c2

            lax.fori_loop(0, K1_BATCH, row, 0)
            base = wid * K1_EPW + it * K1_BATCH
            pltpu.sync_copy(abuf[slot], g_hbm.at[pl.ds(base, K1_BATCH)])

            @pl.when(it + 2 < K1_ITERS)
            def _(slot=slot, it=it):
                issue(slot, it + 2)
        return carry

    lax.fori_loop(0, K1_ROUNDS, round_body, 0)


def _edge_gather_add(src, dst, xi, xj):
    return pl.kernel(
        _k1_body,
        out_type=jax.ShapeDtypeStruct((N_EDGES_C, HID_C), jnp.float32),
        mesh=_sc_mesh(),
        scratch_types=[
            pltpu.VMEM((K1_BATCH,), jnp.int32),
            pltpu.VMEM((K1_BATCH,), jnp.int32),
            pltpu.VMEM((K1_BATCH,), jnp.int32),
            pltpu.VMEM((K1_BATCH,), jnp.int32),
            pltpu.VMEM((K1_BATCH, HID_C), jnp.float32),
            pltpu.VMEM((K1_BATCH, HID_C), jnp.float32),
            pltpu.VMEM((K1_BATCH, HID_C), jnp.float32),
            pltpu.VMEM((K1_BATCH, HID_C), jnp.float32),
            pltpu.SemaphoreType.DMA,
            pltpu.SemaphoreType.DMA,
        ],
    )(src, dst, xi, xj)


def _fwd_body(ha_ref, hb_ref, da_ref, db_ref, w_ref, b_ref, o_ref):
    x = _norm_x(ha_ref, hb_ref, da_ref, db_ref)
    y = jnp.dot(x, w_ref[...], preferred_element_type=jnp.float32) + b_ref[...]
    o_ref[...] = jnp.maximum(y, 0.0)


def _fwd_matmul(ha, hb, da, db, w, b):
    n, _ = ha.shape
    grid = (n // NODE_BLK,)
    hspec = pl.BlockSpec((NODE_BLK, HID_C), lambda i: (i, 0))
    dspec = pl.BlockSpec((1, 1, NODE_BLK), lambda i: (i, 0, 0))
    return pl.pallas_call(
        _fwd_body,
        grid=grid,
        in_specs=[
            hspec, hspec, dspec, dspec,
            pl.BlockSpec((HID_C, HID_C), lambda i: (0, 0)),
            pl.BlockSpec((1, HID_C), lambda i: (0, 0)),
        ],
        out_specs=pl.BlockSpec((NODE_BLK, HID_C), lambda i: (i, 0)),
        out_shape=jax.ShapeDtypeStruct((n, HID_C), jnp.float32),
    )(ha, hb, da, db, w, b)


def _node3_body1(h_ref, w_ref, xi_ref, xj_ref, xn_ref):
    out = jnp.dot(h_ref[...], w_ref[...], preferred_element_type=jnp.float32)
    xi_ref[...] = out[:, :HID_C]
    xj_ref[...] = out[:, HID_C:2 * HID_C]
    xn_ref[...] = out[:, 2 * HID_C:]


def _norm_x(ha_ref, hb_ref, da_ref, db_ref):
    d = (da_ref[...] + db_ref[...]).reshape(NODE_BLK)
    s = jnp.where(d > 0.0, 1.0 / d, 0.0)[:, None]
    return jnp.maximum((ha_ref[...] + hb_ref[...]) * s, 0.0)


def _node3_body2(ha_ref, hb_ref, da_ref, db_ref, w_ref, xi_ref, xj_ref, xn_ref):
    x = _norm_x(ha_ref, hb_ref, da_ref, db_ref)
    out = jnp.dot(x, w_ref[...], preferred_element_type=jnp.float32)
    xi_ref[...] = out[:, :HID_C]
    xj_ref[...] = out[:, HID_C:2 * HID_C]
    xn_ref[...] = out[:, 2 * HID_C:]


def _node_matmul3(hs, w_cat):
    n = hs[0].shape[0]
    grid = (n // NODE_BLK,)
    hspec = pl.BlockSpec((NODE_BLK, HID_C), lambda i: (i, 0))
    dspec = pl.BlockSpec((1, 1, NODE_BLK), lambda i: (i, 0, 0))
    ospec = pl.BlockSpec((NODE_BLK, HID_C), lambda i: (i, 0))
    oshape = jax.ShapeDtypeStruct((n, HID_C), jnp.float32)
    if len(hs) == 1:
        body = _node3_body1
        specs = [hspec]
    else:
        body = _node3_body2
        specs = [hspec, hspec, dspec, dspec]
    return pl.pallas_call(
        body,
        grid=grid,
        in_specs=specs + [pl.BlockSpec((HID_C, 3 * HID_C), lambda i: (0, 0))],
        out_specs=[ospec, ospec, ospec],
        out_shape=[oshape, oshape, oshape],
    )(*hs, w_cat)


def _edge_body(e_ref, g_ref, w_ref, b_ref, a_ref, fo_ref, lo_ref, bm_ref):
    ef = jnp.dot(e_ref[...], w_ref[...], preferred_element_type=jnp.float32)
    f = ef + g_ref[...] + b_ref[...]
    fo = jnp.where(f >= 0.0, f, 0.2 * f)
    fo_ref[...] = fo
    lo = fo * a_ref[...]
    lo = jnp.sum(lo, axis=1)
    lo_ref[...] = lo.reshape(1, 1, EDGE_BLK)
    bm_ref[...] = jnp.max(lo.reshape(EDGE_BLK // HID_C, HID_C), axis=0).reshape(1, 1, HID_C)


def _edge_kernel(e, g, w, b, attn):
    grid = (N_EDGES_C // EDGE_BLK,)
    return pl.pallas_call(
        _edge_body,
        grid=grid,
        in_specs=[
            pl.BlockSpec((EDGE_BLK, HID_C), lambda i: (i, 0)),
            pl.BlockSpec((EDGE_BLK, HID_C), lambda i: (i, 0)),
            pl.BlockSpec((HID_C, HID_C), lambda i: (0, 0)),
            pl.BlockSpec((1, HID_C), lambda i: (0, 0)),
            pl.BlockSpec((1, HID_C), lambda i: (0, 0)),
        ],
        out_specs=[
            pl.BlockSpec((EDGE_BLK, HID_C), lambda i: (i, 0)),
            pl.BlockSpec((1, 1, EDGE_BLK), lambda i: (i, 0, 0)),
            pl.BlockSpec((1, 1, HID_C), lambda i: (i, 0, 0)),
        ],
        out_shape=[
            jax.ShapeDtypeStruct((N_EDGES_C, HID_C), jnp.float32),
            jax.ShapeDtypeStruct((N_EDGES_C // EDGE_BLK, 1, EDGE_BLK), jnp.float32),
            jax.ShapeDtypeStruct((N_EDGES_C // EDGE_BLK, 1, HID_C), jnp.float32),
        ],
    )(e, g, w, b, attn)


EC_C = N_EDGES_C // EDGE_BLK
ND_C = 10
NBLK_C = N_NODES_C // ND_C
DEN_SUB = 256


def _ex_body(lo_ref, bm_ref, ex_ref):
    big_l = jnp.max(bm_ref[...])
    ex_ref[...] = jnp.exp(lo_ref[...] - big_l)


def _ex_kernel(lo3d, bmax):
    return pl.pallas_call(
        _ex_body,
        grid=(EC_C,),
        in_specs=[
            pl.BlockSpec((1, 1, EDGE_BLK), lambda i: (i, 0, 0)),
            pl.BlockSpec((EC_C, 1, HID_C), lambda i: (0, 0, 0)),
        ],
        out_specs=pl.BlockSpec((1, 1, EDGE_BLK), lambda i: (i, 0, 0)),
        out_shape=jax.ShapeDtypeStruct((EC_C, 1, EDGE_BLK), jnp.float32),
    )(lo3d, bmax)


K3_BATCH = 80
K3_ITERS = K1_EPW // K3_BATCH
K3_ROUNDS = (K3_ITERS + 1) // 2


def _k3_body(src_hbm, dst_hbm, ex_hbm, xn_hbm, hpart_hbm, hden_hbm,
             srcv0, srcv1, dstv0, dstv1, exv0, exv1, r0, r1,
             zbuf, zdbuf, sg0, sg1, shared, shden):
    cid = lax.axis_index("c")
    sid = lax.axis_index("s")
    wid = sid * SC_NC + cid
    srcv = (srcv0, srcv1)
    dstv = (dstv0, dstv1)
    exv = (exv0, exv1)
    rows = (r0, r1)
    sg = (sg0, sg1)

    def zero16(r, carry):
        for j in range(HID_C // 16):
            zbuf[r, pl.ds(j * 16, 16)] = jnp.zeros((16,), jnp.float32)
        return carry

    lax.fori_loop(0, zbuf.shape[0], zero16, 0)

    def zero1d(q, carry):
        zdbuf[pl.ds(q * 16, 16)] = jnp.zeros((16,), jnp.float32)
        return carry

    lax.fori_loop(0, 2000 // 16, zero1d, 0)

    @pl.when(sid < 5)
    def _():
        for k in range(20):
            pltpu.sync_copy(zbuf, shared.at[pl.ds(sid * 2000 + k * 100, 100)])
        pltpu.sync_copy(zdbuf, shden.at[pl.ds(sid * 2000, 2000)])

    plsc.subcore_barrier()

    def issue(slot, it):
        base = wid * K1_EPW + it * K3_BATCH
        pltpu.sync_copy(src_hbm.at[pl.ds(base, K3_BATCH)], srcv[slot])
        pltpu.sync_copy(dst_hbm.at[pl.ds(base, K3_BATCH)], dstv[slot])
        pltpu.sync_copy(ex_hbm.at[pl.ds(base, K3_BATCH)], exv[slot])
        pltpu.async_copy(xn_hbm.at[srcv[slot]], rows[slot], sg[slot])

    issue(0, 0)
    issue(1, 1)

    def round_body(r, carry):
        for slot in (0, 1):
            it = r * 2 + slot

            @pl.when(it < K3_ITERS)
            def _(slot=slot, it=it):
                pltpu.make_async_copy(xn_hbm.at[srcv[slot]], rows[slot],
                                      sg[slot]).wait()

                def scale_group(q, c2, slot=slot):
                    a16 = exv[slot][pl.ds(q * 16, 16)]
                    for l in range(16):
                        s = a16[l]
                        rr = q * 16 + l
                        for j in range(HID_C // 16):
                            sl2 = pl.ds(j * 16, 16)
                            rows[slot][rr, sl2] = rows[slot][rr, sl2] * s
                    return c2

                lax.fori_loop(0, K3_BATCH // 16, scale_group, 0)
                pltpu.sync_copy(rows[slot], shared.at[dstv[slot]], add=True)
                pltpu.sync_copy(exv[slot], shden.at[dstv[slot]], add=True)

                @pl.when(it + 2 < K3_ITERS)
                def _(slot=slot, it=it):
                    issue(slot, it + 2)
        return carry

    lax.fori_loop(0, K3_ROUNDS, round_body, 0)
    plsc.subcore_barrier()

    @pl.when(sid < 5)
    def _():
        pltpu.sync_copy(
            shared.at[pl.ds(sid * 2000, 2000)],
            hpart_hbm.at[cid, pl.ds(sid * 2000, 2000)])
        pltpu.sync_copy(shden.at[pl.ds(sid * 2000, 2000)], zdbuf)
        pltpu.sync_copy(
            zdbuf, hden_hbm.at[pl.ds(cid * N_NODES_C + sid * 2000, 2000)])


def _aggregate(src, dst, ex, xn):
    return pl.kernel(
        _k3_body,
        out_type=(
            jax.ShapeDtypeStruct((SC_NC, N_NODES_C, HID_C), jnp.float32),
            jax.ShapeDtypeStruct((SC_NC * N_NODES_C,), jnp.float32),
        ),
        mesh=_sc_mesh(),
        scratch_types=[
            pltpu.VMEM((K3_BATCH,), jnp.int32),
            pltpu.VMEM((K3_BATCH,), jnp.int32),
            pltpu.VMEM((K3_BATCH,), jnp.int32),
            pltpu.VMEM((K3_BATCH,), jnp.int32),
            pltpu.VMEM((K3_BATCH,), jnp.float32),
            pltpu.VMEM((K3_BATCH,), jnp.float32),
            pltpu.VMEM((K3_BATCH, HID_C), jnp.float32),
            pltpu.VMEM((K3_BATCH, HID_C), jnp.float32),
            pltpu.VMEM((100, HID_C), jnp.float32),
            pltpu.VMEM((2000,), jnp.float32),
            pltpu.SemaphoreType.DMA,
            pltpu.SemaphoreType.DMA,
            pltpu.VMEM_SHARED((N_NODES_C, HID_C), jnp.float32),
            pltpu.VMEM_SHARED((N_NODES_C,), jnp.float32),
        ],
    )(src, dst, ex, xn)


def kernel(h_tokens, e_tokens, edge_index, fg_edge_index, token_emb, e_token_emb, W_ni, W_nj, W_fij, egat_bias, egat_attn, W_node, W_fwd, b_fwd, gat_W, gat_attn_l, gat_attn_r, gat_bias, W_lin, b_lin, W_cls, b_cls):
    src, dst = edge_index[0], edge_index[1]
    h = jax.nn.relu(token_emb[h_tokens])
    e = e_token_emb[e_tokens]

    w_cat = jnp.concatenate([W_ni, W_nj, W_node], axis=2)

    hs = (h,)
    for i in range(N_LAYERS_C):
        xi, xj, xn = _node_matmul3(hs, w_cat[i])
        g = _edge_gather_add(src, dst, xi, xj)
        f_out, lo3d, bmax = _edge_kernel(e, g, W_fij[i], egat_bias[i][None, :],
                                         egat_attn[i][None, :])
        e = f_out
        ex3d = _ex_kernel(lo3d, bmax)
        hpart, hden = _aggregate(src, dst, ex3d.reshape(-1), xn)
        hd = hden.reshape(SC_NC, N_NODES_C // NODE_BLK, 1, NODE_BLK)
        hs = (hpart[0], hpart[1], hd[0], hd[1])

    h = _fwd_matmul(hs[0], hs[1], hs[2], hs[3], W_fwd, b_fwd[None, :])

    hs = jnp.sort(h, axis=-1)
    hb = hs.reshape(B_GRAPHS_C, NPG_C, HID_C)
    order = jnp.argsort(-hb[:, :, -1], axis=1)[:, :K_C]
    pooled = jnp.take_along_axis(hb, order[:, :, None], axis=1).reshape(B_GRAPHS_C, K_C * HID_C)

    ft = pooled @ gat_W
    el = jnp.sum(ft * gat_attn_l, axis=-1)
    er = jnp.sum(ft * gat_attn_r, axis=-1)
    fsrc, fdst = fg_edge_index[0], fg_edge_index[1]
    lg = jax.nn.leaky_relu(el[fsrc] + er[fdst], negative_slope=0.2)
    m = jax.ops.segment_max(lg, fdst, num_segments=B_GRAPHS_C)
    ex = jnp.exp(lg - m[fdst])
    den = jax.ops.segment_sum(ex, fdst, num_segments=B_GRAPHS_C)
    a = ex / den[fdst]
    rst = jax.ops.segment_sum(ft[fsrc] * a[:, None], fdst, num_segments=B_GRAPHS_C) + gat_bias
    h = jax.nn.relu(rst)
    h = jax.nn.relu(h @ W_lin + b_lin)
    out = h @ W_cls + b_cls
    return out.reshape(-1, 2)
